# baseline (speedup 1.0000x reference)
import jax
import jax.numpy as jnp
from jax import lax
from jax.experimental import pallas as pl
from jax.experimental.pallas import tpu as pltpu
from jax.experimental.pallas import tpu_sc as plsc

NATM = 384
NOPS = 8
NS = 4
R2 = 0.4 * 0.4
NCHUNK = NATM // 16


def _wrap(d):
    return jnp.where(d > 0.5, d - 1.0, jnp.where(d < -0.5, d + 1.0, d))


def _floor(x):
    t = x.astype(jnp.int32).astype(jnp.float32)
    return jnp.where(x < t, t - 1.0, t)


def _sc_body(xs_hbm, ti_hbm, tf_hbm, out_hbm, xs_v, f1_v, ti_v, tf_v, st_v):
    cid = lax.axis_index("c")
    sid = lax.axis_index("s")
    u = sid * 2 + cid
    pltpu.sync_copy(ti_hbm.at[u], ti_v)
    pltpu.sync_copy(tf_hbm.at[u], tf_v)
    tiv = ti_v[...]
    natm = tiv[0]
    s = tiv[1]
    pltpu.sync_copy(xs_hbm.at[s], xs_v)
    tfv = tf_v[...]

    for ch in range(NCHUNK):
        sl = pl.ds(ch * 16, 16)
        x = xs_v[0, sl]
        y = xs_v[1, sl]
        z = xs_v[2, sl]
        for c in range(3):
            t = (tfv[1 + 3 * c + 0] * x + tfv[1 + 3 * c + 1] * y
                 + tfv[1 + 3 * c + 2] * z)
            f1_v[c, sl] = t - _floor(t)

    lane = lax.broadcasted_iota(jnp.int32, (16,), 0)
    zero16 = jnp.zeros((16,), jnp.float32)
    nch = (natm + 15) // 16

    def pair(buf):
        def ci_body(ci, accs):
            ibase = ci * 16
            ux16 = buf[0, pl.ds(ibase, 16)]
            uy16 = buf[1, pl.ds(ibase, 16)]
            uz16 = buf[2, pl.ds(ibase, 16)]
            for l in range(16):
                ux = jnp.broadcast_to(ux16[l], (16,))
                uy = jnp.broadcast_to(uy16[l], (16,))
                uz = jnp.broadcast_to(uz16[l], (16,))
                limit = jnp.where(ibase + l < natm, natm, 0)

                def cj_body(cj, accs2):
                    ax, ay, az = accs2
                    sl = pl.ds(cj * 16, 16)
                    mx = _wrap(buf[0, sl] - ux)
                    my = _wrap(buf[1, sl] - uy)
                    mz = _wrap(buf[2, sl] - uz)
                    sx = mx * mx
                    sy = my * my
                    sz = mz * mz
                    d2 = sx + sy + sz
                    ok = (d2 <= R2) & (cj * 16 + lane < limit)
                    return (ax + jnp.where(ok, sx, zero16),
                            ay + jnp.where(ok, sy, zero16),
                            az + jnp.where(ok, sz, zero16))

                accs = lax.fori_loop(0, nch, cj_body, accs)
            return accs

        return lax.fori_loop(0, nch, ci_body, (zero16, zero16, zero16))

    o0 = pair(xs_v)
    o1 = pair(f1_v)
    for c in range(3):
        st_v[pl.ds(c * 16, 16)] = o0[c]
        st_v[pl.ds(48 + c * 16, 16)] = o1[c]
    pltpu.sync_copy(st_v, out_hbm.at[u])


def _combine_body(p_ref, w_ref, o_ref):
    p = p_ref[...]

    def csum(base):
        return jnp.sum(p[:, base:base + 16], axis=1, keepdims=True)

    dx = csum(48) - csum(0)
    dy = csum(64) - csum(16)
    dz = csum(80) - csum(32)
    n = jnp.sqrt(dx * dx + dy * dy + dz * dz)
    o_ref[0, 0] = jnp.sum(w_ref[...] * n)


def kernel(fracs, natms, oprss, noprs):
    natms = natms.reshape(-1).astype(jnp.int32)
    noprs = noprs.reshape(-1).astype(jnp.int32)
    fa = jnp.cumsum(natms) - natms
    oa = jnp.cumsum(noprs) - noprs

    frT = jnp.pad(fracs, ((0, NATM), (0, 0))).T
    Xs = jax.vmap(
        lambda st: jax.lax.dynamic_slice(frT, (0, st), (3, NATM)))(fa)

    jidx = jnp.arange(NOPS, dtype=jnp.int32)[None, :]
    opidx = jnp.clip(oa[:, None] + jidx, 0, oprss.shape[0] - 1)
    ops = oprss[opidx]
    w = jnp.where(
        jidx < noprs[:, None],
        1.0 / (jnp.maximum(noprs, 1)[:, None].astype(jnp.float32) * NS),
        0.0).astype(jnp.float32)

    uu = jnp.arange(NS * NOPS, dtype=jnp.int32)
    us = uu // NOPS
    uj = uu % NOPS
    ti = jnp.zeros((NS * NOPS, 16), jnp.int32)
    ti = ti.at[:, 0].set(natms[us]).at[:, 1].set(us)
    tf = jnp.zeros((NS * NOPS, 16), jnp.float32)
    tf = tf.at[:, 0].set(w[us, uj])
    tf = tf.at[:, 1:10].set(ops[us, uj].reshape(NS * NOPS, 9))

    mesh = plsc.VectorSubcoreMesh(core_axis_name="c", subcore_axis_name="s")
    partial = pl.kernel(
        _sc_body,
        out_type=jax.ShapeDtypeStruct((NS * NOPS, 96), jnp.float32),
        mesh=mesh,
        scratch_types=[
            pltpu.VMEM((3, NATM), jnp.float32),
            pltpu.VMEM((3, NATM), jnp.float32),
            pltpu.VMEM((16,), jnp.int32),
            pltpu.VMEM((16,), jnp.float32),
            pltpu.VMEM((96,), jnp.float32),
        ],
    )(Xs, ti, tf)

    out = pl.pallas_call(
        _combine_body,
        out_specs=pl.BlockSpec(memory_space=pltpu.SMEM),
        out_shape=jax.ShapeDtypeStruct((1, 1), jnp.float32),
    )(partial, w.reshape(NS * NOPS, 1))
    return out[0, 0]


# speedup vs baseline: 1.0264x; 1.0264x over previous
import jax
import jax.numpy as jnp
from jax import lax
from jax.experimental import pallas as pl
from jax.experimental.pallas import tpu as pltpu
from jax.experimental.pallas import tpu_sc as plsc

NATM = 384
NOPS = 8
NS = 4
NU = NS * (NOPS + 1)
NW = 32
R2 = 0.4 * 0.4
NCHUNK = NATM // 16


def _wrap(d):
    return jnp.where(d > 0.5, d - 1.0, jnp.where(d < -0.5, d + 1.0, d))


def _floor(x):
    t = x.astype(jnp.int32).astype(jnp.float32)
    return jnp.where(x < t, t - 1.0, t)


def _sc_body(xs_hbm, ti_hbm, tf_hbm, out_hbm, xs_v, f1_v, ti_v, tf_v, st_v):
    cid = lax.axis_index("c")
    sid = lax.axis_index("s")
    w = sid * 2 + cid
    pltpu.sync_copy(xs_hbm, xs_v)
    pltpu.sync_copy(ti_hbm, ti_v)
    pltpu.sync_copy(tf_hbm, tf_v)
    lane = lax.broadcasted_iota(jnp.int32, (16,), 0)
    zero16 = jnp.zeros((16,), jnp.float32)

    def unit_body(u, k):
        tiu = ti_v[pl.ds(u * 16, 16)]
        natm = tiu[0]
        s = tiu[1]
        nch = (natm + 15) // 16
        ci = jnp.remainder(w - k, NW)

        @pl.when(ci < nch)
        def _():
            tfu = tf_v[pl.ds(u * 16, 16)]
            for ch in range(NCHUNK):
                x = xs_v[pl.ds((s * 3 + 0) * NATM + ch * 16, 16)]
                y = xs_v[pl.ds((s * 3 + 1) * NATM + ch * 16, 16)]
                z = xs_v[pl.ds((s * 3 + 2) * NATM + ch * 16, 16)]
                for c in range(3):
                    t = tfu[3 * c + 0] * x + tfu[3 * c + 1] * y \
                        + tfu[3 * c + 2] * z
                    f1_v[pl.ds(c * NATM + ch * 16, 16)] = t - _floor(t)
            ux16 = f1_v[pl.ds(0 * NATM + ci * 16, 16)]
            uy16 = f1_v[pl.ds(1 * NATM + ci * 16, 16)]
            uz16 = f1_v[pl.ds(2 * NATM + ci * 16, 16)]
            ibase = ci * 16
            us = []
            for l in range(16):
                us.append((jnp.broadcast_to(ux16[l], (16,)),
                           jnp.broadcast_to(uy16[l], (16,)),
                           jnp.broadcast_to(uz16[l], (16,)),
                           jnp.where(ibase + l < natm, natm, 0)))

            def cj_body(cj, accs):
                ax, ay, az = accs
                jj = cj * 16 + lane
                xj = f1_v[pl.ds(0 * NATM + cj * 16, 16)]
                yj = f1_v[pl.ds(1 * NATM + cj * 16, 16)]
                zj = f1_v[pl.ds(2 * NATM + cj * 16, 16)]
                for l in range(16):
                    ux, uy, uz, limit = us[l]
                    mx = _wrap(xj - ux)
                    my = _wrap(yj - uy)
                    mz = _wrap(zj - uz)
                    sx = mx * mx
                    sy = my * my
                    sz = mz * mz
                    d2 = sx + sy + sz
                    ok = (d2 <= R2) & (jj < limit)
                    ax = ax + jnp.where(ok, sx, zero16)
                    ay = ay + jnp.where(ok, sy, zero16)
                    az = az + jnp.where(ok, sz, zero16)
                return ax, ay, az

            ax, ay, az = lax.fori_loop(0, nch, cj_body,
                                       (zero16, zero16, zero16))
            st_v[pl.ds(0, 16)] = ax
            st_v[pl.ds(16, 16)] = ay
            st_v[pl.ds(32, 16)] = az
            pltpu.sync_copy(st_v, out_hbm.at[u, w])

        return k + nch

    lax.fori_loop(0, NU, unit_body, jnp.int32(0))


def _combine_body(p_ref, m_ref, w_ref, o_ref):
    p = jnp.where(m_ref[...] > 0, p_ref[...], 0.0)
    q = jnp.sum(p, axis=1)
    sx = jnp.sum(q[:, 0:16], axis=1, keepdims=True)
    sy = jnp.sum(q[:, 16:32], axis=1, keepdims=True)
    sz = jnp.sum(q[:, 32:48], axis=1, keepdims=True)
    S = jnp.concatenate([sx, sy, sz], axis=1)
    base = S[0:NS, :]
    rest = S[NS:, :]
    tiled = jnp.concatenate([base] * NOPS, axis=0)
    d = rest - tiled
    n2 = jnp.sum(d * d, axis=1, keepdims=True)
    o_ref[0, 0] = jnp.sum(w_ref[...] * jnp.sqrt(n2))


def kernel(fracs, natms, oprss, noprs):
    natms = natms.reshape(-1).astype(jnp.int32)
    noprs = noprs.reshape(-1).astype(jnp.int32)
    fa = jnp.cumsum(natms) - natms
    oa = jnp.cumsum(noprs) - noprs

    frT = jnp.pad(fracs, ((0, NATM), (0, 0))).T
    Xs = jax.vmap(
        lambda st: jax.lax.dynamic_slice(frT, (0, st), (3, NATM)))(fa)

    jidx = jnp.arange(NOPS, dtype=jnp.int32)[None, :]
    opidx = jnp.clip(oa[:, None] + jidx, 0, oprss.shape[0] - 1)
    ops = oprss[opidx]
    wtab = jnp.where(
        jidx < noprs[:, None],
        1.0 / (jnp.maximum(noprs, 1)[:, None].astype(jnp.float32) * NS),
        0.0).astype(jnp.float32)

    uu = jnp.arange(NU, dtype=jnp.int32)
    us = uu % NS
    slot = uu // NS
    active = (slot == 0) | ((slot - 1) < noprs[us])
    natm_eff = jnp.where(active, natms[us], 0)
    ti = jnp.zeros((NU, 16), jnp.int32)
    ti = ti.at[:, 0].set(natm_eff).at[:, 1].set(us)
    eye = jnp.broadcast_to(jnp.eye(3, dtype=jnp.float32), (NU, 3, 3))
    opsu = jnp.where((slot > 0)[:, None, None],
                     ops[us, jnp.maximum(slot - 1, 0)], eye)
    tf = jnp.zeros((NU, 16), jnp.float32)
    tf = tf.at[:, 0:9].set(opsu.reshape(NU, 9))

    nch_u = (natm_eff + 15) // 16
    K_u = jnp.cumsum(nch_u) - nch_u
    ww = jnp.arange(NW, dtype=jnp.int32)[None, :]
    ci_uw = jnp.remainder(ww - K_u[:, None], NW)
    owned = (ci_uw < nch_u[:, None]).astype(jnp.float32)
    mask3 = jnp.broadcast_to(owned[:, :, None], (NU, NW, 48))

    mesh = plsc.VectorSubcoreMesh(core_axis_name="c", subcore_axis_name="s")
    partial = pl.kernel(
        _sc_body,
        out_type=jax.ShapeDtypeStruct((NU, NW, 48), jnp.float32),
        mesh=mesh,
        scratch_types=[
            pltpu.VMEM((NS * 3 * NATM,), jnp.float32),
            pltpu.VMEM((3 * NATM,), jnp.float32),
            pltpu.VMEM((NU * 16,), jnp.int32),
            pltpu.VMEM((NU * 16,), jnp.float32),
            pltpu.VMEM((48,), jnp.float32),
        ],
    )(Xs.reshape(-1), ti.reshape(-1), tf.reshape(-1))

    w32 = wtab.T.reshape(NS * NOPS, 1)
    out = pl.pallas_call(
        _combine_body,
        out_specs=pl.BlockSpec(memory_space=pltpu.SMEM),
        out_shape=jax.ShapeDtypeStruct((1, 1), jnp.float32),
    )(partial, mask3, w32)
    return out[0, 0]


# speedup vs baseline: 1.9991x; 1.9477x over previous
import jax
import jax.numpy as jnp
from jax import lax
from jax.experimental import pallas as pl
from jax.experimental.pallas import tpu as pltpu
from jax.experimental.pallas import tpu_sc as plsc

NATM = 384
NOPS = 8
NS = 4
NU = NS * (NOPS + 1)
NW = 32
R2 = 0.4 * 0.4
NCHUNK = NATM // 16


def _wrap(d):
    return jnp.where(d > 0.5, d - 1.0, jnp.where(d < -0.5, d + 1.0, d))


def _floor(x):
    t = x.astype(jnp.int32).astype(jnp.float32)
    return jnp.where(x < t, t - 1.0, t)


def _sc_body(xs_hbm, ti_hbm, tf_hbm, out_hbm, xs_v, f1_v, ti_v, tf_v, st_v):
    cid = lax.axis_index("c")
    sid = lax.axis_index("s")
    w = sid * 2 + cid
    pltpu.sync_copy(xs_hbm, xs_v)
    pltpu.sync_copy(ti_hbm, ti_v)
    pltpu.sync_copy(tf_hbm, tf_v)
    lane = lax.broadcasted_iota(jnp.int32, (16,), 0)
    zero16 = jnp.zeros((16,), jnp.float32)

    def unit_body(u, k):
        tiu = ti_v[pl.ds(u * 16, 16)]
        natm = tiu[0]
        s = tiu[1]
        nch = (natm + 15) // 16
        ci = jnp.remainder(w - k, NW)

        @pl.when(ci < nch)
        def _():
            tfu = tf_v[pl.ds(u * 16, 16)]
            for ch in range(NCHUNK):
                x = xs_v[pl.ds((s * 3 + 0) * NATM + ch * 16, 16)]
                y = xs_v[pl.ds((s * 3 + 1) * NATM + ch * 16, 16)]
                z = xs_v[pl.ds((s * 3 + 2) * NATM + ch * 16, 16)]
                for c in range(3):
                    t = tfu[3 * c + 0] * x + tfu[3 * c + 1] * y \
                        + tfu[3 * c + 2] * z
                    f1_v[pl.ds(c * NATM + ch * 16, 16)] = t - _floor(t)
            ux16 = f1_v[pl.ds(0 * NATM + ci * 16, 16)]
            uy16 = f1_v[pl.ds(1 * NATM + ci * 16, 16)]
            uz16 = f1_v[pl.ds(2 * NATM + ci * 16, 16)]
            ibase = ci * 16
            accs = (zero16, zero16, zero16)
            for g in range(4):
                rows = []
                for l in range(4):
                    li = g * 4 + l
                    rows.append((jnp.broadcast_to(ux16[li], (16,)),
                                 jnp.broadcast_to(uy16[li], (16,)),
                                 jnp.broadcast_to(uz16[li], (16,)),
                                 jnp.where(ibase + li < natm, natm, 0)))

                def cj_body(cj, accs2):
                    ax, ay, az = accs2
                    jj = cj * 16 + lane
                    xj = f1_v[pl.ds(0 * NATM + cj * 16, 16)]
                    yj = f1_v[pl.ds(1 * NATM + cj * 16, 16)]
                    zj = f1_v[pl.ds(2 * NATM + cj * 16, 16)]
                    for ux, uy, uz, limit in rows:
                        adx = jnp.abs(xj - ux)
                        ady = jnp.abs(yj - uy)
                        adz = jnp.abs(zj - uz)
                        tx = jnp.minimum(adx, 1.0 - adx)
                        ty = jnp.minimum(ady, 1.0 - ady)
                        tz = jnp.minimum(adz, 1.0 - adz)
                        sx = tx * tx
                        sy = ty * ty
                        sz = tz * tz
                        d2 = sx + sy + sz
                        ok = (d2 <= R2) & (jj < limit)
                        ax = ax + jnp.where(ok, sx, zero16)
                        ay = ay + jnp.where(ok, sy, zero16)
                        az = az + jnp.where(ok, sz, zero16)
                    return ax, ay, az

                accs = lax.fori_loop(0, nch, cj_body, accs)
            ax, ay, az = accs
            st_v[pl.ds(0, 16)] = ax
            st_v[pl.ds(16, 16)] = ay
            st_v[pl.ds(32, 16)] = az
            pltpu.sync_copy(st_v, out_hbm.at[u, w])

        return k + nch

    lax.fori_loop(0, NU, unit_body, jnp.int32(0))


def _combine_body(p_ref, m_ref, w_ref, o_ref):
    p = jnp.where(m_ref[...] > 0, p_ref[...], 0.0)
    q = jnp.sum(p, axis=1)
    sx = jnp.sum(q[:, 0:16], axis=1, keepdims=True)
    sy = jnp.sum(q[:, 16:32], axis=1, keepdims=True)
    sz = jnp.sum(q[:, 32:48], axis=1, keepdims=True)
    S = jnp.concatenate([sx, sy, sz], axis=1)
    base = S[0:NS, :]
    rest = S[NS:, :]
    tiled = jnp.concatenate([base] * NOPS, axis=0)
    d = rest - tiled
    n2 = jnp.sum(d * d, axis=1, keepdims=True)
    o_ref[0, 0] = jnp.sum(w_ref[...] * jnp.sqrt(n2))


def kernel(fracs, natms, oprss, noprs):
    natms = natms.reshape(-1).astype(jnp.int32)
    noprs = noprs.reshape(-1).astype(jnp.int32)
    fa = jnp.cumsum(natms) - natms
    oa = jnp.cumsum(noprs) - noprs

    frT = jnp.pad(fracs, ((0, NATM), (0, 0))).T
    Xs = jax.vmap(
        lambda st: jax.lax.dynamic_slice(frT, (0, st), (3, NATM)))(fa)

    jidx = jnp.arange(NOPS, dtype=jnp.int32)[None, :]
    opidx = jnp.clip(oa[:, None] + jidx, 0, oprss.shape[0] - 1)
    ops = oprss[opidx]
    wtab = jnp.where(
        jidx < noprs[:, None],
        1.0 / (jnp.maximum(noprs, 1)[:, None].astype(jnp.float32) * NS),
        0.0).astype(jnp.float32)

    uu = jnp.arange(NU, dtype=jnp.int32)
    us = uu % NS
    slot = uu // NS
    active = (slot == 0) | ((slot - 1) < noprs[us])
    natm_eff = jnp.where(active, natms[us], 0)
    ti = jnp.zeros((NU, 16), jnp.int32)
    ti = ti.at[:, 0].set(natm_eff).at[:, 1].set(us)
    eye = jnp.broadcast_to(jnp.eye(3, dtype=jnp.float32), (NU, 3, 3))
    opsu = jnp.where((slot > 0)[:, None, None],
                     ops[us, jnp.maximum(slot - 1, 0)], eye)
    tf = jnp.zeros((NU, 16), jnp.float32)
    tf = tf.at[:, 0:9].set(opsu.reshape(NU, 9))

    nch_u = (natm_eff + 15) // 16
    K_u = jnp.cumsum(nch_u) - nch_u
    ww = jnp.arange(NW, dtype=jnp.int32)[None, :]
    ci_uw = jnp.remainder(ww - K_u[:, None], NW)
    owned = (ci_uw < nch_u[:, None]).astype(jnp.float32)
    mask3 = owned[:, :, None]

    mesh = plsc.VectorSubcoreMesh(core_axis_name="c", subcore_axis_name="s")
    partial = pl.kernel(
        _sc_body,
        out_type=jax.ShapeDtypeStruct((NU, NW, 48), jnp.float32),
        mesh=mesh,
        scratch_types=[
            pltpu.VMEM((NS * 3 * NATM,), jnp.float32),
            pltpu.VMEM((3 * NATM,), jnp.float32),
            pltpu.VMEM((NU * 16,), jnp.int32),
            pltpu.VMEM((NU * 16,), jnp.float32),
            pltpu.VMEM((48,), jnp.float32),
        ],
    )(Xs.reshape(-1), ti.reshape(-1), tf.reshape(-1))

    w32 = wtab.T.reshape(NS * NOPS, 1)
    out = pl.pallas_call(
        _combine_body,
        out_specs=pl.BlockSpec(memory_space=pltpu.SMEM),
        out_shape=jax.ShapeDtypeStruct((1, 1), jnp.float32),
    )(partial, mask3, w32)
    return out[0, 0]


# speedup vs baseline: 2.0428x; 1.0219x over previous
import jax
import jax.numpy as jnp
from jax import lax
from jax.experimental import pallas as pl
from jax.experimental.pallas import tpu as pltpu
from jax.experimental.pallas import tpu_sc as plsc

NTOT = 1536
NATM = 384
NOPS = 8
NS = 4
NU = NS * (NOPS + 1)
NW = 32
R2 = 0.4 * 0.4


def _floor(x):
    t = x.astype(jnp.int32).astype(jnp.float32)
    return jnp.where(x < t, t - 1.0, t)


def _sc_body(xs_hbm, ti_hbm, tf_hbm, out_hbm, xs_v, f1_v, ti_v, tf_v, st_v):
    cid = lax.axis_index("c")
    sid = lax.axis_index("s")
    w = sid * 2 + cid
    pltpu.sync_copy(xs_hbm, xs_v)
    pltpu.sync_copy(ti_hbm, ti_v)
    pltpu.sync_copy(tf_hbm, tf_v)
    lane = lax.broadcasted_iota(jnp.int32, (16,), 0)
    zero16 = jnp.zeros((16,), jnp.float32)

    def unit_body(u, k):
        tiu = ti_v[pl.ds(u * 16, 16)]
        natm = tiu[0]
        fa = tiu[1]
        ca = tiu[2]
        nch = tiu[3]
        ci = jnp.remainder(w - k, NW)

        @pl.when(ci < nch)
        def _():
            tfu = tf_v[pl.ds(u * 16, 16)]

            def stage(ch, carry):
                gb = (ca + ch) * 16
                x = xs_v[pl.ds(0 * NTOT + gb, 16)]
                y = xs_v[pl.ds(1 * NTOT + gb, 16)]
                z = xs_v[pl.ds(2 * NTOT + gb, 16)]
                for c in range(3):
                    t = tfu[3 * c + 0] * x + tfu[3 * c + 1] * y \
                        + tfu[3 * c + 2] * z
                    f1_v[pl.ds(c * NTOT + gb, 16)] = t - _floor(t)
                return carry

            lax.fori_loop(0, nch, stage, 0)

            ib = (ca + ci) * 16
            ux16 = f1_v[pl.ds(0 * NTOT + ib, 16)]
            uy16 = f1_v[pl.ds(1 * NTOT + ib, 16)]
            uz16 = f1_v[pl.ds(2 * NTOT + ib, 16)]
            accs = (zero16, zero16, zero16)
            for g in range(4):
                rows = []
                for l in range(4):
                    li = g * 4 + l
                    gi = ib + li
                    ok_row = (gi >= fa) & (gi < fa + natm)
                    lim = jnp.where(ok_row, natm, 0).astype(jnp.uint32)
                    rows.append((jnp.broadcast_to(ux16[li], (16,)),
                                 jnp.broadcast_to(uy16[li], (16,)),
                                 jnp.broadcast_to(uz16[li], (16,)),
                                 lim))

                def cj_body(cj, accs2):
                    ax, ay, az = accs2
                    jb = (ca + cj) * 16
                    jd = (jb + lane - fa).astype(jnp.uint32)
                    xj = f1_v[pl.ds(0 * NTOT + jb, 16)]
                    yj = f1_v[pl.ds(1 * NTOT + jb, 16)]
                    zj = f1_v[pl.ds(2 * NTOT + jb, 16)]
                    for ux, uy, uz, lim in rows:
                        adx = jnp.abs(xj - ux)
                        ady = jnp.abs(yj - uy)
                        adz = jnp.abs(zj - uz)
                        tx = jnp.minimum(adx, 1.0 - adx)
                        ty = jnp.minimum(ady, 1.0 - ady)
                        tz = jnp.minimum(adz, 1.0 - adz)
                        sx = tx * tx
                        sy = ty * ty
                        sz = tz * tz
                        d2 = sx + sy + sz
                        ok = (d2 <= R2) & (jd < lim)
                        ax = ax + jnp.where(ok, sx, zero16)
                        ay = ay + jnp.where(ok, sy, zero16)
                        az = az + jnp.where(ok, sz, zero16)
                    return ax, ay, az

                accs = lax.fori_loop(0, nch, cj_body, accs)
            ax, ay, az = accs
            st_v[pl.ds(0, 16)] = ax
            st_v[pl.ds(16, 16)] = ay
            st_v[pl.ds(32, 16)] = az
            pltpu.sync_copy(st_v, out_hbm.at[u, w])

        return k + nch

    lax.fori_loop(0, NU, unit_body, jnp.int32(0))


def _combine_body(p_ref, m_ref, w_ref, o_ref):
    p = jnp.where(m_ref[...] > 0, p_ref[...], 0.0)
    q = jnp.sum(p, axis=1)
    sx = jnp.sum(q[:, 0:16], axis=1, keepdims=True)
    sy = jnp.sum(q[:, 16:32], axis=1, keepdims=True)
    sz = jnp.sum(q[:, 32:48], axis=1, keepdims=True)
    S = jnp.concatenate([sx, sy, sz], axis=1)
    base = S[0:NS, :]
    rest = S[NS:, :]
    tiled = jnp.concatenate([base] * NOPS, axis=0)
    d = rest - tiled
    n2 = jnp.sum(d * d, axis=1, keepdims=True)
    o_ref[0, 0] = jnp.sum(w_ref[...] * jnp.sqrt(n2))


def kernel(fracs, natms, oprss, noprs):
    natms = natms.reshape(-1).astype(jnp.int32)
    noprs = noprs.reshape(-1).astype(jnp.int32)
    fa = jnp.cumsum(natms) - natms
    oa = jnp.cumsum(noprs) - noprs
    fb = fa + natms

    ca_s = fa // 16
    nch_s = jnp.where(natms > 0, (fb - 1) // 16 - ca_s + 1, 0)

    jidx = jnp.arange(NOPS, dtype=jnp.int32)[None, :]
    opidx = jnp.clip(oa[:, None] + jidx, 0, oprss.shape[0] - 1)
    ops = oprss[opidx]
    wtab = jnp.where(
        jidx < noprs[:, None],
        1.0 / (jnp.maximum(noprs, 1)[:, None].astype(jnp.float32) * NS),
        0.0).astype(jnp.float32)

    uu = jnp.arange(NU, dtype=jnp.int32)
    us = uu % NS
    slot = uu // NS
    active = (slot == 0) | ((slot - 1) < noprs[us])
    natm_eff = jnp.where(active, natms[us], 0)
    nch_u = jnp.where(active, nch_s[us], 0)
    zc = jnp.zeros((NU,), jnp.int32)
    ti = jnp.stack([natm_eff, fa[us], ca_s[us], nch_u]
                   + [zc] * 12, axis=1)
    eye = jnp.broadcast_to(jnp.eye(3, dtype=jnp.float32), (NU, 3, 3))
    opsu = jnp.where((slot > 0)[:, None, None],
                     ops[us, jnp.maximum(slot - 1, 0)], eye)
    tf = jnp.concatenate([opsu.reshape(NU, 9),
                          jnp.zeros((NU, 7), jnp.float32)], axis=1)

    K_u = jnp.cumsum(nch_u) - nch_u
    ww = jnp.arange(NW, dtype=jnp.int32)[None, :]
    ci_uw = jnp.remainder(ww - K_u[:, None], NW)
    owned = (ci_uw < nch_u[:, None]).astype(jnp.float32)
    mask3 = owned[:, :, None]

    mesh = plsc.VectorSubcoreMesh(core_axis_name="c", subcore_axis_name="s")
    partial = pl.kernel(
        _sc_body,
        out_type=jax.ShapeDtypeStruct((NU, NW, 48), jnp.float32),
        mesh=mesh,
        scratch_types=[
            pltpu.VMEM((3 * NTOT,), jnp.float32),
            pltpu.VMEM((3 * NTOT,), jnp.float32),
            pltpu.VMEM((NU * 16,), jnp.int32),
            pltpu.VMEM((NU * 16,), jnp.float32),
            pltpu.VMEM((48,), jnp.float32),
        ],
    )(fracs.T.reshape(-1), ti.reshape(-1), tf.reshape(-1))

    w32 = wtab.T.reshape(NS * NOPS, 1)
    out = pl.pallas_call(
        _combine_body,
        out_specs=pl.BlockSpec(memory_space=pltpu.SMEM),
        out_shape=jax.ShapeDtypeStruct((1, 1), jnp.float32),
    )(partial, mask3, w32)
    return out[0, 0]
